# baseline (speedup 1.0000x reference)
import functools

import jax
import jax.numpy as jnp
from jax import lax
from jax.experimental import pallas as pl
from jax.experimental.pallas import tpu as pltpu

_LANE = 128
_INT_MIN = -2147483648
_MASK31 = 0x7FFFFFFF
_NEG_INF = float("-inf")


def _body(v_ref, k_ref, p_ref, o_ref, e_ref):
    v = v_ref[...]
    m = jnp.max(v, axis=1, keepdims=True)
    e = jnp.exp(v - m)
    e_ref[...] = e
    kv = k_ref[:, :1]
    pv = p_ref[:, :1]

    vp = v.shape[1]
    csz = 13 * 128

    def _rowsum(x):
        parts = [
            jnp.sum(x[:, j:min(j + csz, vp)], axis=1, keepdims=True)
            for j in range(0, vp, csz)
        ]
        tot = parts[0]
        for q in parts[1:]:
            tot = tot + q
        return tot

    def _key_to_f32(ck):
        ck = jnp.where(ck >= 0, ck, jnp.maximum(ck, -2139095041))
        fb = jnp.where(ck >= 0, ck, ck ^ _MASK31)
        return lax.bitcast_convert_type(fb, jnp.float32)

    def cnt_ge(cand):
        cf = _key_to_f32(cand)
        return _rowsum((v_ref[...] >= cf).astype(jnp.int32))

    zero = jnp.zeros_like(kv)
    base = jnp.where(cnt_ge(zero) >= kv, 0, _INT_MIN)

    def step1(i, rem):
        bit = jnp.left_shift(jnp.int32(1), 30 - i)
        cand = base + (rem | bit)
        return jnp.where(cnt_ge(cand) >= kv, rem | bit, rem)

    tkf = _key_to_f32(base + lax.fori_loop(0, 31, step1, zero))

    s = _rowsum(jnp.where(v_ref[...] >= tkf, e_ref[...], 0.0))
    ps = pv * s

    def mass_gt(cand):
        ev = e_ref[...]
        eb = lax.bitcast_convert_type(ev, jnp.int32)
        return _rowsum(jnp.where(eb > cand, ev, 0.0))

    def step2(i, rem):
        bit = jnp.left_shift(jnp.int32(1), 29 - i)
        cand = rem | bit
        return jnp.where(mass_gt(cand) >= ps, rem | bit, rem)

    mp = lax.fori_loop(0, 30, step2, zero)

    eb = lax.bitcast_convert_type(e_ref[...], jnp.int32)
    keep = (v >= tkf) & ((eb > mp) | (v == m))
    o_ref[...] = jnp.where(keep, v, _NEG_INF)


@functools.partial(jax.jit, static_argnames=())
def kernel(logits, k, p):
    bsz, vocab = logits.shape
    vp = pl.cdiv(vocab, _LANE) * _LANE
    rblk = 16
    logits = logits.astype(jnp.float32)
    if vp != vocab:
        pad = jnp.full((bsz, vp - vocab), _NEG_INF, jnp.float32)
        lp = jnp.concatenate([logits, pad], axis=1)
    else:
        lp = logits
    kb = jnp.broadcast_to(
        jnp.clip(k.astype(jnp.int32), 1, vocab)[:, None], (bsz, _LANE))
    pb = jnp.broadcast_to(p.astype(jnp.float32)[:, None], (bsz, _LANE))
    out = pl.pallas_call(
        _body,
        grid=(bsz // rblk,),
        in_specs=[
            pl.BlockSpec((rblk, vp), lambda i: (i, 0)),
            pl.BlockSpec((rblk, _LANE), lambda i: (i, 0)),
            pl.BlockSpec((rblk, _LANE), lambda i: (i, 0)),
        ],
        out_specs=pl.BlockSpec((rblk, vp), lambda i: (i, 0)),
        out_shape=jax.ShapeDtypeStruct((bsz, vp), jnp.float32),
        scratch_shapes=[
            pltpu.VMEM((rblk, vp), jnp.float32),
        ],
    )(lp, kb, pb)
    return out[:, :vocab]
